# Initial kernel scaffold; baseline (speedup 1.0000x reference)
import jax
import jax.numpy as jnp
from jax.experimental import pallas as pl


def kernel(x, edge_index, W1, a_src1, a_dst1, b1, W2, a_src2, a_dst2, b2):
    raise NotImplementedError("write your pallas kernel here")


# speedup vs baseline: 49.5349x; 49.5349x over previous
import functools

import jax
import jax.numpy as jnp
from jax import lax
from jax.experimental import pallas as pl
from jax.experimental.pallas import tpu as pltpu
from jax.experimental.pallas import tpu_sc as plsc

_NC = 2
_NS = 16
_NW = _NC * _NS
_K = 128
_BR = 128
_D = 64
_DP = 2 * _D


def _bcast_attn(a):
    H, C = a.shape
    eye = jnp.eye(H, dtype=a.dtype)
    blk = a[:, :, None, None] * eye[:, None, :, None]
    blk = jnp.broadcast_to(blk, (H, C, H, C))
    return blk.reshape(H * C, H * C)


def _prep1_body(x_ref, w_ref, as_ref, ad_ref, g_ref, d_ref):
    h = jnp.dot(x_ref[...], w_ref[...], preferred_element_type=jnp.float32)
    bs = jnp.dot(h, as_ref[...], preferred_element_type=jnp.float32)
    bd = jnp.dot(h, ad_ref[...], preferred_element_type=jnp.float32)
    g_ref[...] = jnp.concatenate([h, bs], axis=1)
    d_ref[...] = jnp.concatenate([bd, bd], axis=1)


def _mid_body(acc_ref, b_ref, w_ref, as_ref, ad_ref, g_ref, d_ref):
    s = acc_ref[0] + acc_ref[1]
    h1 = s[:, :_D] / (s[:, _D:] + 1e-16) + b_ref[...]
    h1 = jnp.where(h1 > 0, h1, jnp.exp(jnp.minimum(h1, 0.0)) - 1.0)
    h2 = jnp.dot(h1, w_ref[...], preferred_element_type=jnp.float32)
    bs = jnp.dot(h2, as_ref[...], preferred_element_type=jnp.float32)
    bd = jnp.dot(h2, ad_ref[...], preferred_element_type=jnp.float32)
    g_ref[...] = jnp.concatenate([h2, bs], axis=1)
    d_ref[...] = jnp.concatenate([bd, bd], axis=1)


def _final_body(acc_ref, b_ref, o_ref):
    s = acc_ref[0] + acc_ref[1]
    o_ref[...] = s[:, :_D] / (s[:, _D:] + 1e-16) + b_ref[...]


@functools.lru_cache(maxsize=None)
def _make_edge_kernel(npad, epw):
    stripe = npad // _NS
    nchunks = epw // _K
    mesh = plsc.VectorSubcoreMesh(core_axis_name="c", subcore_axis_name="s",
                                  num_cores=_NC, num_subcores=_NS)

    @functools.partial(
        pl.kernel,
        out_type=jax.ShapeDtypeStruct((_NC, npad, _DP), jnp.float32),
        mesh=mesh,
        scratch_types=[
            pltpu.VMEM((_K,), jnp.int32),
            pltpu.VMEM((_K,), jnp.int32),
            pltpu.VMEM((_K, _DP), jnp.float32),
            pltpu.VMEM((_K, _DP), jnp.float32),
            pltpu.VMEM((_K, _DP), jnp.float32),
            pltpu.VMEM((_DP,), jnp.float32),
            pltpu.VMEM_SHARED((npad, _DP), jnp.float32),
            pltpu.SemaphoreType.DMA,
            pltpu.SemaphoreType.DMA,
        ],
    )
    def edge_kernel(src_hbm, dst_hbm, g_hbm, d_hbm, cv_hbm, z_hbm,
                    acc_out,
                    sidx, didx, gv, dv, mwv, cvv,
                    acc_sh, sem0, sem1):
        cid = lax.axis_index("c")
        sid = lax.axis_index("s")
        wid = sid * _NC + cid
        r0 = sid * stripe
        pltpu.sync_copy(z_hbm, acc_sh.at[pl.ds(r0, stripe)])
        pltpu.sync_copy(cv_hbm, cvv)
        plsc.subcore_barrier()

        cvs = [cvv[pl.ds(k * 16, 16)] for k in range(_D // 16)]
        ebase = wid * epw

        @pl.loop(0, nchunks)
        def _chunk(g):
            base = ebase + g * _K
            pltpu.sync_copy(src_hbm.at[pl.ds(base, _K)], sidx)
            pltpu.sync_copy(dst_hbm.at[pl.ds(base, _K)], didx)
            ga = pltpu.async_copy(g_hbm.at[sidx], gv, sem0)
            gb = pltpu.async_copy(d_hbm.at[didx], dv, sem1)
            ga.wait()
            gb.wait()

            @pl.loop(0, _K)
            def _edge(i):
                for k in range(_D // 16):
                    lo = pl.ds(k * 16, 16)
                    hi = pl.ds(_D + k * 16, 16)
                    e = gv[i, hi] + dv[i, lo]
                    l = jnp.maximum(e, e * 0.2)
                    w = jnp.exp(l - cvs[k])
                    mwv[i, lo] = w * gv[i, lo]
                    mwv[i, hi] = w

            pltpu.sync_copy(mwv, acc_sh.at[didx], add=True)

        plsc.subcore_barrier()
        pltpu.sync_copy(acc_sh.at[pl.ds(r0, stripe)],
                        acc_out.at[cid, pl.ds(r0, stripe)])

    return edge_kernel


def _tc_prep1(x_pad, W1, As1, Ad1, npad, f_in):
    grid = (npad // _BR,)
    full = lambda shp: pl.BlockSpec(shp, lambda i: (0, 0))
    return pl.pallas_call(
        _prep1_body,
        grid=grid,
        in_specs=[pl.BlockSpec((_BR, f_in), lambda i: (i, 0)),
                  full((f_in, _D)), full((_D, _D)), full((_D, _D))],
        out_specs=[pl.BlockSpec((_BR, _DP), lambda i: (i, 0))] * 2,
        out_shape=[jax.ShapeDtypeStruct((npad, _DP), jnp.float32)] * 2,
    )(x_pad, W1, As1, Ad1)


def _tc_mid(acc1, b1, W2, As2, Ad2, npad):
    grid = (npad // _BR,)
    big = pl.BlockSpec((_NC, _BR, _DP), lambda i: (0, i, 0))
    full = lambda shp: pl.BlockSpec(shp, lambda i: (0, 0))
    return pl.pallas_call(
        _mid_body,
        grid=grid,
        in_specs=[big, full((1, _D)),
                  full((_D, _D)), full((_D, _D)), full((_D, _D))],
        out_specs=[pl.BlockSpec((_BR, _DP), lambda i: (i, 0))] * 2,
        out_shape=[jax.ShapeDtypeStruct((npad, _DP), jnp.float32)] * 2,
    )(acc1, b1.reshape(1, _D), W2, As2, Ad2)


def _tc_final(acc2, b2, npad):
    grid = (npad // _BR,)
    big = pl.BlockSpec((_NC, _BR, _DP), lambda i: (0, i, 0))
    full = lambda shp: pl.BlockSpec(shp, lambda i: (0, 0))
    return pl.pallas_call(
        _final_body,
        grid=grid,
        in_specs=[big, full((1, _D))],
        out_specs=pl.BlockSpec((_BR, _D), lambda i: (i, 0)),
        out_shape=jax.ShapeDtypeStruct((npad, _D), jnp.float32),
    )(acc2, b2.reshape(1, _D))


def kernel(x, edge_index, W1, a_src1, a_dst1, b1, W2, a_src2, a_dst2, b2):
    N, F_in = x.shape
    E = edge_index.shape[1]
    ET = E + N

    npad = ((N + 1 + _BR - 1) // _BR) * _BR
    stripe = npad // _NS

    epw = ((ET + _NW * _K - 1) // (_NW * _K)) * _K
    epad = _NW * epw

    loops = jnp.arange(N, dtype=jnp.int32)
    src = jnp.full((epad,), N, jnp.int32)
    src = src.at[:E].set(edge_index[0].astype(jnp.int32)).at[E:ET].set(loops)
    dst = jnp.full((epad,), N, jnp.int32)
    dst = dst.at[:E].set(edge_index[1].astype(jnp.int32)).at[E:ET].set(loops)

    x_pad = jnp.zeros((npad, F_in), jnp.float32).at[:N].set(x)
    zeros = jnp.zeros((stripe, _DP), jnp.float32)

    As1 = _bcast_attn(a_src1)
    Ad1 = _bcast_attn(a_dst1)
    As2 = _bcast_attn(a_src2)
    Ad2 = _bcast_attn(a_dst2)

    edge_kernel = _make_edge_kernel(npad, epw)

    def cpad(g, d):
        c = jnp.max(g[:, _D:], axis=0) + jnp.max(d[:, :_D], axis=0)
        return jnp.concatenate([c, jnp.zeros((_DP - _D,), jnp.float32)])

    G1, D1 = _tc_prep1(x_pad, W1, As1, Ad1, npad, F_in)
    acc1 = edge_kernel(src, dst, G1, D1, cpad(G1, D1), zeros)

    G2, D2 = _tc_mid(acc1, b1, W2, As2, Ad2, npad)
    acc2 = edge_kernel(src, dst, G2, D2, cpad(G2, D2), zeros)

    out = _tc_final(acc2, b2, npad)
    return out[:N]
